# baseline (speedup 1.0000x reference)
import functools

import jax
import jax.numpy as jnp
from jax import lax
from jax.experimental import pallas as pl
from jax.experimental.pallas import tpu as pltpu
from jax.experimental.pallas import tpu_sc as plsc

_N_TABLE = 100000
_D = 128
_S = 32
_B = 10000

_ROWS_BLK = 2000

_NW = 32
_CN = 4
_NCH2 = 160
_NCH_F = 120
_NCH_S = _NCH2 - _NCH_F
_BPAD = 16 * _NCH2 * _CN


def _tc_body(x_ref, wt_ref, b_ref, o_ref):
    o_ref[...] = jnp.maximum(
        jnp.dot(x_ref[...], wt_ref[...], preferred_element_type=jnp.float32)
        + b_ref[...],
        0.0,
    )


def _transform_table(feat_table, Wt, b2):
    return pl.pallas_call(
        _tc_body,
        grid=(_N_TABLE // _ROWS_BLK,),
        in_specs=[
            pl.BlockSpec((_ROWS_BLK, _D), lambda i: (i, 0)),
            pl.BlockSpec((_D, _D), lambda i: (0, 0)),
            pl.BlockSpec((1, _D), lambda i: (0, 0)),
        ],
        out_specs=pl.BlockSpec((_ROWS_BLK, _D), lambda i: (i, 0)),
        out_shape=jax.ShapeDtypeStruct((_N_TABLE, _D), jnp.float32),
    )(feat_table, Wt, b2)


@functools.partial(
    pl.kernel,
    mesh=plsc.VectorSubcoreMesh(core_axis_name="c", subcore_axis_name="s"),
    out_type=jax.ShapeDtypeStruct((_BPAD, _D), jnp.float32),
    scratch_types=[
        pltpu.VMEM((_NCH_F, 128), jnp.int32),
        pltpu.VMEM((4, 128, _D), jnp.float32),
        pltpu.VMEM((2 * _CN, _D), jnp.float32),
        pltpu.SemaphoreType.DMA,
        pltpu.SemaphoreType.DMA,
        pltpu.SemaphoreType.DMA,
        pltpu.SemaphoreType.DMA,
    ],
)
def _sc_pool(idx_hbm, t_hbm, out_hbm, idx_v, ring, acc_v, s0, s1, s2, s3):
    c = lax.axis_index("c")
    s = lax.axis_index("s")
    base = s * _NCH2 + c * _NCH_F
    count = jnp.where(c == 0, _NCH_F, _NCH_S)
    pltpu.sync_copy(idx_hbm.at[pl.ds(base, _NCH_F)], idx_v)
    sems = (s0, s1, s2, s3)

    def fire(j, b):
        pltpu.async_copy(t_hbm.at[idx_v.at[j]], ring.at[b], sems[b])

    for b in range(3):
        fire(b, b)

    def quad(p, carry):
        for bsel in range(4):
            j = 4 * p + bsel

            @pl.when(j + 3 < count)
            def _():
                fire(j + 3, (bsel + 3) % 4)

            pltpu.make_async_copy(
                t_hbm.at[pl.ds(0, 128)], ring.at[bsel], sems[bsel]
            ).wait()
            half = (bsel % 2) * _CN
            for i in range(_CN):
                for g in range(_D // 16):
                    acc = ring[bsel, i * _S, pl.ds(g * 16, 16)]
                    for smp in range(1, _S):
                        acc = acc + ring[bsel, i * _S + smp, pl.ds(g * 16, 16)]
                    acc_v[half + i, pl.ds(g * 16, 16)] = acc * (1.0 / _S)
            if bsel % 2 == 1:
                off = pl.multiple_of((base + j - 1) * _CN, 8)
                pltpu.sync_copy(acc_v, out_hbm.at[pl.ds(off, 2 * _CN)])
        return carry

    lax.fori_loop(0, count // 4, quad, 0)


def kernel(nodes, to_neighs, feat_table, W, b):
    del nodes
    t = _transform_table(feat_table, W.T, b.reshape(1, _D))
    idx = jnp.pad(to_neighs.astype(jnp.int32), ((0, _BPAD - _B), (0, 0)))
    idx_rows = idx.reshape(_BPAD * _S // 128, 128)
    idx_rows = jnp.pad(idx_rows, ((0, _NCH_F - _NCH_S), (0, 0)))
    out = _sc_pool(idx_rows, t)
    return out[:_B]


# speedup vs baseline: 1.0075x; 1.0075x over previous
import functools

import jax
import jax.numpy as jnp
from jax import lax
from jax.experimental import pallas as pl
from jax.experimental.pallas import tpu as pltpu
from jax.experimental.pallas import tpu_sc as plsc

_N_TABLE = 100000
_D = 128
_S = 32
_B = 10000

_ROWS_BLK = 2000

_NW = 32
_CN = 4
_NCH2 = 160
_NCH_F = 112
_NCH_S = _NCH2 - _NCH_F
_BPAD = 16 * _NCH2 * _CN


def _tc_body(x_ref, wt_ref, b_ref, o_ref):
    o_ref[...] = jnp.maximum(
        jnp.dot(x_ref[...], wt_ref[...], preferred_element_type=jnp.float32)
        + b_ref[...],
        0.0,
    )


def _transform_table(feat_table, Wt, b2):
    return pl.pallas_call(
        _tc_body,
        grid=(_N_TABLE // _ROWS_BLK,),
        in_specs=[
            pl.BlockSpec((_ROWS_BLK, _D), lambda i: (i, 0)),
            pl.BlockSpec((_D, _D), lambda i: (0, 0)),
            pl.BlockSpec((1, _D), lambda i: (0, 0)),
        ],
        out_specs=pl.BlockSpec((_ROWS_BLK, _D), lambda i: (i, 0)),
        out_shape=jax.ShapeDtypeStruct((_N_TABLE, _D), jnp.float32),
    )(feat_table, Wt, b2)


@functools.partial(
    pl.kernel,
    mesh=plsc.VectorSubcoreMesh(core_axis_name="c", subcore_axis_name="s"),
    out_type=jax.ShapeDtypeStruct((_BPAD, _D), jnp.float32),
    scratch_types=[
        pltpu.VMEM((_NCH_F, 128), jnp.int32),
        pltpu.VMEM((4, 128, _D), jnp.float32),
        pltpu.VMEM((2 * _CN, _D), jnp.float32),
        pltpu.SemaphoreType.DMA,
        pltpu.SemaphoreType.DMA,
        pltpu.SemaphoreType.DMA,
        pltpu.SemaphoreType.DMA,
    ],
)
def _sc_pool(idx_hbm, t_hbm, out_hbm, idx_v, ring, acc_v, s0, s1, s2, s3):
    c = lax.axis_index("c")
    s = lax.axis_index("s")
    base = s * _NCH2 + c * _NCH_F
    count = jnp.where(c == 0, _NCH_F, _NCH_S)
    pltpu.sync_copy(idx_hbm.at[pl.ds(base, _NCH_F)], idx_v)
    sems = (s0, s1, s2, s3)

    def fire(j, b):
        pltpu.async_copy(
            t_hbm.at[idx_v.at[j, pl.ds(0, 64)]],
            ring.at[b, pl.ds(0, 64)],
            sems[b],
        )
        pltpu.async_copy(
            t_hbm.at[idx_v.at[j, pl.ds(64, 64)]],
            ring.at[b, pl.ds(64, 64)],
            sems[b],
        )

    for b in range(3):
        fire(b, b)

    def quad(p, carry):
        for bsel in range(4):
            j = 4 * p + bsel

            @pl.when(j + 3 < count)
            def _():
                fire(j + 3, (bsel + 3) % 4)

            pltpu.make_async_copy(
                t_hbm.at[pl.ds(0, 128)], ring.at[bsel], sems[bsel]
            ).wait()
            half = (bsel % 2) * _CN
            for i in range(_CN):
                for g in range(_D // 16):
                    acc = ring[bsel, i * _S, pl.ds(g * 16, 16)]
                    for smp in range(1, _S):
                        acc = acc + ring[bsel, i * _S + smp, pl.ds(g * 16, 16)]
                    acc_v[half + i, pl.ds(g * 16, 16)] = acc * (1.0 / _S)
            if bsel % 2 == 1:
                off = pl.multiple_of((base + j - 1) * _CN, 8)
                pltpu.sync_copy(acc_v, out_hbm.at[pl.ds(off, 2 * _CN)])
        return carry

    lax.fori_loop(0, count // 4, quad, 0)


def kernel(nodes, to_neighs, feat_table, W, b):
    del nodes
    t = _transform_table(feat_table, W.T, b.reshape(1, _D))
    idx = jnp.pad(to_neighs.astype(jnp.int32), ((0, _BPAD - _B), (0, 0)))
    idx_rows = idx.reshape(_BPAD * _S // 128, 128)
    idx_rows = jnp.pad(idx_rows, ((0, _NCH_F - _NCH_S), (0, 0)))
    out = _sc_pool(idx_rows, t)
    return out[:_B]
